# baseline (speedup 1.0000x reference)
import jax
import jax.numpy as jnp
from jax import lax
from jax.experimental import pallas as pl
from jax.experimental.pallas import tpu as pltpu
from jax.experimental.pallas import tpu_sc as plsc

NCLASS = 100
DIM = 128
BATCH = 4096
NPAD = 128

_NC = 2
_NS = 16
_NW = _NC * _NS
_BPW = BATCH // _NW

_CB = 100
_MSTEPS = NCLASS // _CB
_ROWS = 256
_NBLK = BATCH // _ROWS


def _sc_gather_body(table_hbm, idx_hbm, out_hbm, idx_v, rows_v, sem):
    wid = lax.axis_index("s") * _NC + lax.axis_index("c")
    base = wid * _BPW
    pltpu.sync_copy(idx_hbm.at[pl.ds(base, _BPW)], idx_v)
    pltpu.async_copy(table_hbm.at[idx_v], rows_v, sem).wait()
    pltpu.sync_copy(rows_v, out_hbm.at[pl.ds(base, _BPW)])


def _sc_scatter_body(rows_hbm, idx_hbm, out_hbm, idx_v, rows_v, sem):
    wid = lax.axis_index("s") * _NC + lax.axis_index("c")
    base = wid * _BPW
    pltpu.sync_copy(idx_hbm.at[pl.ds(base, _BPW)], idx_v)
    pltpu.sync_copy(rows_hbm.at[pl.ds(base, _BPW)], rows_v)
    pltpu.async_copy(rows_v, out_hbm.at[idx_v], sem).wait()


def _sc_scatter(rows, idx):
    mesh = plsc.VectorSubcoreMesh(
        core_axis_name="c", subcore_axis_name="s",
        num_cores=_NC, num_subcores=_NS,
    )
    return pl.kernel(
        _sc_scatter_body,
        out_type=jax.ShapeDtypeStruct((BATCH, NPAD), jnp.float32),
        mesh=mesh,
        scratch_types=[
            pltpu.VMEM((_BPW,), jnp.int32),
            pltpu.VMEM((_BPW, NPAD), jnp.float32),
            pltpu.SemaphoreType.DMA,
        ],
    )(rows, idx)


def _sc_gather(table, idx):
    mesh = plsc.VectorSubcoreMesh(
        core_axis_name="c", subcore_axis_name="s",
        num_cores=_NC, num_subcores=_NS,
    )
    return pl.kernel(
        _sc_gather_body,
        out_type=jax.ShapeDtypeStruct((BATCH, NPAD), jnp.float32),
        mesh=mesh,
        scratch_types=[
            pltpu.VMEM((_BPW,), jnp.int32),
            pltpu.VMEM((_BPW, NPAD), jnp.float32),
            pltpu.SemaphoreType.DMA,
        ],
    )(table, idx)


def _fused_kernel(isc_ref, first_ref, last_ref, ord_ref, w_ref, tsort_ref,
                  st_ref, out_ref, m_ref, blk_ref):
    i = pl.program_id(0)

    @pl.when(i < _MSTEPS)
    def _():
        for k in range(_CB):
            c = i * _CB + k
            w0 = w_ref[k, 0]
            w1 = w_ref[k, 1]
            a = lax.dot_general(
                w0, w1, (((1,), (1,)), ((), ())),
                preferred_element_type=jnp.float32,
            )
            rows = lax.broadcasted_iota(jnp.int32, (NCLASS, NCLASS), 0)
            cols = lax.broadcasted_iota(jnp.int32, (NCLASS, NCLASS), 1)
            a = a + jnp.where(rows == cols, isc_ref[c], jnp.float32(0.0))
            e = jnp.exp(a)
            denom = jnp.sum(e, axis=0, keepdims=True)
            m = e * (jnp.float32(1.0) / denom)
            m_ref[c] = jnp.pad(m, ((0, NPAD - NCLASS), (0, NPAD - NCLASS)))

    @pl.when(i >= _MSTEPS)
    def _():
        j = i - _MSTEPS
        p = tsort_ref[...]
        lo = first_ref[j]
        hi = last_ref[j]

        UNROLL = 4
        iters = (hi - lo + UNROLL) // UNROLL

        def body(q, acc):
            base_c = lo + q * UNROLL
            for k in range(UNROLL):
                c = base_c + k
                m_c = m_ref[jnp.minimum(c, NCLASS - 1)]
                pm = jnp.dot(p, m_c, preferred_element_type=jnp.float32)
                mask = st_ref[...] == c
                acc = jnp.where(mask, pm, acc)
            return acc

        acc = lax.fori_loop(0, iters, body, jnp.zeros((_ROWS, NPAD), jnp.float32))
        s = jnp.sum(p, axis=1, keepdims=True)
        blk_ref[...] = acc / s
        for r in range(_ROWS):
            idx = ord_ref[j * _ROWS + r]
            out_ref[pl.ds(idx, 1), :] = blk_ref[pl.ds(r, 1), :NCLASS]


def kernel(TeacherLogit, Target, W, Iscale):
    iota = lax.iota(jnp.int32, BATCH)
    st, order = lax.sort((Target, iota), num_keys=1)
    st_blocks = st.reshape(_NBLK, _ROWS)
    first = st_blocks[:, 0]
    last = st_blocks[:, -1]

    tl_pad = jnp.pad(TeacherLogit, ((0, 0), (0, NPAD - NCLASS)))
    tsort = _sc_gather(tl_pad, order)

    out = pl.pallas_call(
        _fused_kernel,
        grid=(_MSTEPS + _NBLK,),
        in_specs=[
            pl.BlockSpec(memory_space=pltpu.SMEM),
            pl.BlockSpec(memory_space=pltpu.SMEM),
            pl.BlockSpec(memory_space=pltpu.SMEM),
            pl.BlockSpec(memory_space=pltpu.SMEM),
            pl.BlockSpec((_CB, 2, NCLASS, DIM),
                         lambda i: (jnp.minimum(i, _MSTEPS - 1), 0, 0, 0)),
            pl.BlockSpec((_ROWS, NPAD),
                         lambda i: (jnp.clip(i - _MSTEPS, 0, _NBLK - 1), 0)),
            pl.BlockSpec((_ROWS, 1),
                         lambda i: (jnp.clip(i - _MSTEPS, 0, _NBLK - 1), 0)),
        ],
        out_specs=pl.BlockSpec((BATCH, NCLASS), lambda i: (0, 0)),
        out_shape=jax.ShapeDtypeStruct((BATCH, NCLASS), jnp.float32),
        scratch_shapes=[
            pltpu.VMEM((NCLASS, NPAD, NPAD), jnp.float32),
            pltpu.VMEM((_ROWS, NPAD), jnp.float32),
        ],
    )(Iscale, first, last, order, W, tsort, st.reshape(BATCH, 1))
    return out


# speedup vs baseline: 1.0298x; 1.0298x over previous
import jax
import jax.numpy as jnp
from jax import lax
from jax.experimental import pallas as pl
from jax.experimental.pallas import tpu as pltpu
from jax.experimental.pallas import tpu_sc as plsc

NCLASS = 100
DIM = 128
BATCH = 4096
NPAD = 128

_NC = 2
_NS = 16
_NW = _NC * _NS
_BPW = BATCH // _NW

_CB = 50
_MSTEPS = NCLASS // _CB
_ROWS = 256
_NBLK = BATCH // _ROWS


def _sc_gather_body(table_hbm, idx_hbm, out_hbm, idx_v, rows_v, sem):
    wid = lax.axis_index("s") * _NC + lax.axis_index("c")
    base = wid * _BPW
    pltpu.sync_copy(idx_hbm.at[pl.ds(base, _BPW)], idx_v)
    pltpu.async_copy(table_hbm.at[idx_v], rows_v, sem).wait()
    pltpu.sync_copy(rows_v, out_hbm.at[pl.ds(base, _BPW)])


def _sc_scatter_body(rows_hbm, idx_hbm, out_hbm, idx_v, rows_v, sem):
    wid = lax.axis_index("s") * _NC + lax.axis_index("c")
    base = wid * _BPW
    pltpu.sync_copy(idx_hbm.at[pl.ds(base, _BPW)], idx_v)
    pltpu.sync_copy(rows_hbm.at[pl.ds(base, _BPW)], rows_v)
    pltpu.async_copy(rows_v, out_hbm.at[idx_v], sem).wait()


def _sc_scatter(rows, idx):
    mesh = plsc.VectorSubcoreMesh(
        core_axis_name="c", subcore_axis_name="s",
        num_cores=_NC, num_subcores=_NS,
    )
    return pl.kernel(
        _sc_scatter_body,
        out_type=jax.ShapeDtypeStruct((BATCH, NPAD), jnp.float32),
        mesh=mesh,
        scratch_types=[
            pltpu.VMEM((_BPW,), jnp.int32),
            pltpu.VMEM((_BPW, NPAD), jnp.float32),
            pltpu.SemaphoreType.DMA,
        ],
    )(rows, idx)


def _sc_gather(table, idx):
    mesh = plsc.VectorSubcoreMesh(
        core_axis_name="c", subcore_axis_name="s",
        num_cores=_NC, num_subcores=_NS,
    )
    return pl.kernel(
        _sc_gather_body,
        out_type=jax.ShapeDtypeStruct((BATCH, NPAD), jnp.float32),
        mesh=mesh,
        scratch_types=[
            pltpu.VMEM((_BPW,), jnp.int32),
            pltpu.VMEM((_BPW, NPAD), jnp.float32),
            pltpu.SemaphoreType.DMA,
        ],
    )(table, idx)


def _fused_kernel(isc_ref, first_ref, last_ref, ord_ref, w_ref, tsort_ref,
                  st_ref, out_ref, m_ref, blk_ref):
    i = pl.program_id(0)

    @pl.when(i < _MSTEPS)
    def _():
        for k in range(_CB):
            c = i * _CB + k
            w0 = w_ref[k, 0]
            w1 = w_ref[k, 1]
            a = lax.dot_general(
                w0, w1, (((1,), (1,)), ((), ())),
                preferred_element_type=jnp.float32,
            )
            rows = lax.broadcasted_iota(jnp.int32, (NCLASS, NCLASS), 0)
            cols = lax.broadcasted_iota(jnp.int32, (NCLASS, NCLASS), 1)
            a = a + jnp.where(rows == cols, isc_ref[c], jnp.float32(0.0))
            e = jnp.exp(a)
            denom = jnp.sum(e, axis=0, keepdims=True)
            m = e * (jnp.float32(1.0) / denom)
            m_ref[c] = jnp.pad(m, ((0, NPAD - NCLASS), (0, NPAD - NCLASS)))

    @pl.when(i >= _MSTEPS)
    def _():
        j = i - _MSTEPS
        p = tsort_ref[...]
        lo = first_ref[j]
        hi = last_ref[j]

        UNROLL = 8
        iters = (hi - lo + UNROLL) // UNROLL

        def body(q, acc):
            base_c = lo + q * UNROLL
            for k in range(UNROLL):
                c = base_c + k
                m_c = m_ref[jnp.minimum(c, NCLASS - 1)]
                pm = jnp.dot(p, m_c, preferred_element_type=jnp.float32)
                mask = st_ref[...] == c
                acc = jnp.where(mask, pm, acc)
            return acc

        acc = lax.fori_loop(0, iters, body, jnp.zeros((_ROWS, NPAD), jnp.float32))
        s = jnp.sum(p, axis=1, keepdims=True)
        blk_ref[...] = acc / s
        for r in range(_ROWS):
            idx = ord_ref[j * _ROWS + r]
            out_ref[pl.ds(idx, 1), :] = blk_ref[pl.ds(r, 1), :NCLASS]


def kernel(TeacherLogit, Target, W, Iscale):
    iota = lax.iota(jnp.int32, BATCH)
    st, order = lax.sort((Target, iota), num_keys=1)
    st_blocks = st.reshape(_NBLK, _ROWS)
    first = st_blocks[:, 0]
    last = st_blocks[:, -1]

    tl_pad = jnp.pad(TeacherLogit, ((0, 0), (0, NPAD - NCLASS)))
    tsort = _sc_gather(tl_pad, order)

    out = pl.pallas_call(
        _fused_kernel,
        grid=(_MSTEPS + _NBLK,),
        in_specs=[
            pl.BlockSpec(memory_space=pltpu.SMEM),
            pl.BlockSpec(memory_space=pltpu.SMEM),
            pl.BlockSpec(memory_space=pltpu.SMEM),
            pl.BlockSpec(memory_space=pltpu.SMEM),
            pl.BlockSpec((_CB, 2, NCLASS, DIM),
                         lambda i: (jnp.minimum(i, _MSTEPS - 1), 0, 0, 0)),
            pl.BlockSpec((_ROWS, NPAD),
                         lambda i: (jnp.clip(i - _MSTEPS, 0, _NBLK - 1), 0)),
            pl.BlockSpec((_ROWS, 1),
                         lambda i: (jnp.clip(i - _MSTEPS, 0, _NBLK - 1), 0)),
        ],
        out_specs=pl.BlockSpec((BATCH, NCLASS), lambda i: (0, 0)),
        out_shape=jax.ShapeDtypeStruct((BATCH, NCLASS), jnp.float32),
        scratch_shapes=[
            pltpu.VMEM((NCLASS, NPAD, NPAD), jnp.float32),
            pltpu.VMEM((_ROWS, NPAD), jnp.float32),
        ],
    )(Iscale, first, last, order, W, tsort, st.reshape(BATCH, 1))
    return out
